# Initial kernel scaffold; baseline (speedup 1.0000x reference)
import jax
import jax.numpy as jnp
from jax.experimental import pallas as pl


def kernel(slot_hidden, temperature, W, b):
    raise NotImplementedError("write your pallas kernel here")


# speedup vs baseline: 1.1493x; 1.1493x over previous
import jax
import jax.numpy as jnp
from jax.experimental import pallas as pl
from jax.experimental.pallas import tpu as pltpu

B = 128
D = 2048
N_NODES = 32
N_CATS = 256
K = 8
NB = 4
GRID = N_NODES // NB


def _fused_kernel(x_ref, t_ref, w_ref, b_ref,
                  score_ref, nodes_ref, cats_ref, l_ref):
    i = pl.program_id(0)
    x = x_ref[...]
    for j in range(NB):
        lt = jax.lax.dot_general(
            x, w_ref[j], (((1,), (0,)), ((), ())),
            preferred_element_type=jnp.float32)
        l_ref[i * NB + j] = lt + b_ref[0, j][None, :]

    @pl.when(i == GRID - 1)
    def _finalize():
        L = l_ref[...]
        inv_t = 1.0 / t_ref[0]
        Lt = L * inv_t
        m = jnp.max(Lt, axis=(0, 2), keepdims=True)
        e = jnp.exp(Lt - m)
        s = jnp.sum(e, axis=(0, 2), keepdims=True)
        p = e / s
        ct = jnp.sum(p, axis=1) * (1.0 / B)
        lm = jnp.sum(L, axis=1) * (1.0 / B)
        ii = (jax.lax.broadcasted_iota(jnp.int32, (N_NODES, N_CATS), 0)
              * N_CATS
              + jax.lax.broadcasted_iota(jnp.int32, (N_NODES, N_CATS), 1))
        work = ct
        score = jnp.float32(0.0)
        big = jnp.int32(2 ** 30)
        for k in range(K):
            mv = jnp.max(work)
            hit = work == mv
            idx = jnp.min(jnp.where(hit, ii, big))
            sel = ii == idx
            score = score + jnp.sum(jnp.where(sel, lm, 0.0))
            work = jnp.where(sel, jnp.float32(-1.0), work)
            nodes_ref[k] = idx // N_CATS
            cats_ref[k] = idx % N_CATS
        score_ref[0] = score


def kernel(slot_hidden, temperature, W, b):
    t = temperature.reshape(1).astype(jnp.float32)
    b3 = b.reshape(GRID, NB, N_CATS)
    score, nodes, cats = pl.pallas_call(
        _fused_kernel,
        grid=(GRID,),
        in_specs=[
            pl.BlockSpec((B, D), lambda i: (0, 0)),
            pl.BlockSpec(memory_space=pltpu.SMEM),
            pl.BlockSpec((NB, D, N_CATS), lambda i: (i, 0, 0)),
            pl.BlockSpec((1, NB, N_CATS), lambda i: (i, 0, 0)),
        ],
        out_specs=[
            pl.BlockSpec(memory_space=pltpu.SMEM),
            pl.BlockSpec(memory_space=pltpu.SMEM),
            pl.BlockSpec(memory_space=pltpu.SMEM),
        ],
        out_shape=[
            jax.ShapeDtypeStruct((1,), jnp.float32),
            jax.ShapeDtypeStruct((K,), jnp.int32),
            jax.ShapeDtypeStruct((K,), jnp.int32),
        ],
        scratch_shapes=[pltpu.VMEM((N_NODES, B, N_CATS), jnp.float32)],
        compiler_params=pltpu.CompilerParams(
            dimension_semantics=("arbitrary",)),
    )(slot_hidden, t, W, b3)
    return (score.reshape(()), nodes, cats)


# speedup vs baseline: 1.1884x; 1.0340x over previous
import jax
import jax.numpy as jnp
from jax.experimental import pallas as pl
from jax.experimental.pallas import tpu as pltpu

B = 128
D = 2048
N_NODES = 32
N_CATS = 256
K = 8
NB = 4
GRID = N_NODES // NB


HNB = NB // 2


def _fused_kernel(x_ref, t_ref, wa_ref, wb_ref, b_ref,
                  score_ref, nodes_ref, cats_ref, l_ref):
    i = pl.program_id(0)
    x = x_ref[...]
    for h, w_ref in ((0, wa_ref), (1, wb_ref)):
        for j in range(HNB):
            lt = jax.lax.dot_general(
                x, w_ref[j], (((1,), (0,)), ((), ())),
                preferred_element_type=jnp.float32)
            l_ref[i * NB + h * HNB + j] = lt + b_ref[0, h * HNB + j][None, :]

    @pl.when(i == GRID - 1)
    def _finalize():
        L = l_ref[...]
        inv_t = 1.0 / t_ref[0]
        Lt = L * inv_t
        m = jnp.max(Lt, axis=(0, 2), keepdims=True)
        e = jnp.exp(Lt - m)
        s = jnp.sum(e, axis=(0, 2), keepdims=True)
        p = e / s
        ct = jnp.sum(p, axis=1) * (1.0 / B)
        lm = jnp.sum(L, axis=1) * (1.0 / B)
        ii = (jax.lax.broadcasted_iota(jnp.int32, (N_NODES, N_CATS), 0)
              * N_CATS
              + jax.lax.broadcasted_iota(jnp.int32, (N_NODES, N_CATS), 1))
        work = ct
        score = jnp.float32(0.0)
        big = jnp.int32(2 ** 30)
        for k in range(K):
            mv = jnp.max(work)
            hit = work == mv
            idx = jnp.min(jnp.where(hit, ii, big))
            sel = ii == idx
            score = score + jnp.sum(jnp.where(sel, lm, 0.0))
            work = jnp.where(sel, jnp.float32(-1.0), work)
            nodes_ref[k] = idx // N_CATS
            cats_ref[k] = idx % N_CATS
        score_ref[0] = score


def kernel(slot_hidden, temperature, W, b):
    t = temperature.reshape(1).astype(jnp.float32)
    b3 = b.reshape(GRID, NB, N_CATS)
    score, nodes, cats = pl.pallas_call(
        _fused_kernel,
        grid=(GRID,),
        in_specs=[
            pl.BlockSpec((B, D), lambda i: (0, 0)),
            pl.BlockSpec(memory_space=pltpu.SMEM),
            pl.BlockSpec((HNB, D, N_CATS), lambda i: (2 * i, 0, 0)),
            pl.BlockSpec((HNB, D, N_CATS), lambda i: (2 * i + 1, 0, 0)),
            pl.BlockSpec((1, NB, N_CATS), lambda i: (i, 0, 0)),
        ],
        out_specs=[
            pl.BlockSpec(memory_space=pltpu.SMEM),
            pl.BlockSpec(memory_space=pltpu.SMEM),
            pl.BlockSpec(memory_space=pltpu.SMEM),
        ],
        out_shape=[
            jax.ShapeDtypeStruct((1,), jnp.float32),
            jax.ShapeDtypeStruct((K,), jnp.int32),
            jax.ShapeDtypeStruct((K,), jnp.int32),
        ],
        scratch_shapes=[pltpu.VMEM((N_NODES, B, N_CATS), jnp.float32)],
        compiler_params=pltpu.CompilerParams(
            dimension_semantics=("arbitrary",)),
    )(slot_hidden, t, W, W, b3)
    return (score.reshape(()), nodes, cats)
